# baseline (speedup 1.0000x reference)
import functools

import jax
import jax.numpy as jnp
from jax import lax
from jax.experimental import pallas as pl
from jax.experimental.pallas import tpu as pltpu
from jax.experimental.pallas import tpu_sc as plsc

_LAMB = max(5.0, 1500.0 / 1.1)
_INV = 1.0 / (1.0 + _LAMB)
_LANES = 16
_WIN = 128


def _sc_gather(cos2, phi2, t32, n_rows, n_cols):
    info = plsc.get_sparse_core_info()
    nw = info.num_cores * info.num_subcores
    bpw = n_rows // nw
    mesh = plsc.VectorSubcoreMesh(core_axis_name="c", subcore_axis_name="s")

    @functools.partial(
        pl.kernel,
        mesh=mesh,
        out_type=(
            jax.ShapeDtypeStruct((n_rows, _WIN), jnp.float32),
            jax.ShapeDtypeStruct((n_rows, _WIN), jnp.float32),
        ),
        scratch_types=[
            pltpu.VMEM((bpw,), jnp.int32),
            pltpu.VMEM((bpw,), jnp.int32),
            pltpu.VMEM((bpw, _WIN), jnp.float32),
            pltpu.SemaphoreType.DMA,
        ],
    )
    def gather_kernel(cos_hbm, phi_hbm, t_hbm, cwin_out, pwin_out,
                      t_v, cidx_v, win_v, sem):
        wid = lax.axis_index("s") * info.num_cores + lax.axis_index("c")
        base = wid * bpw
        pltpu.sync_copy(t_hbm.at[pl.ds(base, bpw)], t_v)
        for j in range(bpw // _LANES):
            tj = t_v[pl.ds(j * _LANES, _LANES)]
            row = base + j * _LANES + lax.iota(jnp.int32, _LANES)
            flat = row * n_cols + tj
            cidx_v[pl.ds(j * _LANES, _LANES)] = flat >> 7
        for src, dst in ((cos_hbm, cwin_out), (phi_hbm, pwin_out)):
            pltpu.async_copy(src.at[cidx_v], win_v, sem).wait()
            pltpu.sync_copy(win_v, dst.at[pl.ds(base, bpw)])

    return gather_kernel(cos2, phi2, t32)


def _tc_body(cos_ref, cwin_ref, pwin_ref, tgt_ref, loss_ref, acc_ref, *, n_rows):
    i = pl.program_id(0)
    n = pl.num_programs(0)
    x = cos_ref[...]
    bm = jnp.max(x, axis=1, keepdims=True)
    s = jnp.sum(jnp.exp(x - bm), axis=1, keepdims=True)
    col = lax.broadcasted_iota(jnp.int32, x.shape, 1)
    cand = jnp.where(x == bm, col, jnp.int32(2**30))
    amax = jnp.min(cand, axis=1, keepdims=True)

    t = tgt_ref[...]
    rb = t.shape[0]
    rglob = i * rb + lax.broadcasted_iota(jnp.int32, (rb, 1), 0)
    lane = lax.broadcasted_iota(jnp.int32, (rb, _WIN), 1)
    ncmod = jnp.int32(x.shape[1] % _WIN)
    fine = lane == ((rglob * ncmod + t) & (_WIN - 1))
    ct = jnp.sum(jnp.where(fine, cwin_ref[...], 0.0), axis=1, keepdims=True)
    pt = jnp.sum(jnp.where(fine, pwin_ref[...], 0.0), axis=1, keepdims=True)
    v = ct + (pt - ct) * _INV
    s_adj = s - jnp.exp(ct - bm) + jnp.exp(v - bm)
    logpt = v - bm - jnp.log(s_adj)
    part_loss = -jnp.sum(logpt, axis=(0, 1), keepdims=True)
    part_acc = jnp.sum((amax == t).astype(jnp.float32), axis=(0, 1), keepdims=True)

    @pl.when(i == 0)
    def _init():
        loss_ref[...] = jnp.zeros((1, 1), jnp.float32)
        acc_ref[...] = jnp.zeros((1, 1), jnp.float32)

    loss_ref[...] += part_loss
    acc_ref[...] += part_acc

    @pl.when(i == n - 1)
    def _finish():
        scale = jnp.float32(1.0 / n_rows)
        loss_ref[...] *= scale
        acc_ref[...] *= scale


def _tc_pass(cos_theta, cwin, pwin, t32, row_block):
    n_rows, n_cols = cos_theta.shape
    grid = (n_rows // row_block,)
    body = functools.partial(_tc_body, n_rows=n_rows)
    return pl.pallas_call(
        body,
        grid=grid,
        in_specs=[
            pl.BlockSpec((row_block, n_cols), lambda i: (i, 0)),
            pl.BlockSpec((row_block, _WIN), lambda i: (i, 0)),
            pl.BlockSpec((row_block, _WIN), lambda i: (i, 0)),
            pl.BlockSpec((row_block, 1), lambda i: (i, 0)),
        ],
        out_specs=(
            pl.BlockSpec((1, 1), lambda i: (0, 0)),
            pl.BlockSpec((1, 1), lambda i: (0, 0)),
        ),
        out_shape=(
            jax.ShapeDtypeStruct((1, 1), jnp.float32),
            jax.ShapeDtypeStruct((1, 1), jnp.float32),
        ),
        compiler_params=pltpu.CompilerParams(
            dimension_semantics=("arbitrary",),
        ),
    )(cos_theta, cwin, pwin, t32)


def kernel(cos_theta, phi_theta, target):
    n_rows, n_cols = cos_theta.shape
    t32 = target.reshape(-1).astype(jnp.int32)
    cos2 = cos_theta.reshape(n_rows * n_cols // _WIN, _WIN)
    phi2 = phi_theta.reshape(n_rows * n_cols // _WIN, _WIN)
    cwin, pwin = _sc_gather(cos2, phi2, t32, n_rows, n_cols)
    loss, acc = _tc_pass(
        cos_theta,
        cwin,
        pwin,
        t32.reshape(n_rows, 1),
        row_block=8,
    )
    return loss[0, 0], acc[0, 0]


# speedup vs baseline: 1.5693x; 1.5693x over previous
import functools

import jax
import jax.numpy as jnp
from jax import lax
from jax.experimental import pallas as pl
from jax.experimental.pallas import tpu as pltpu
from jax.experimental.pallas import tpu_sc as plsc

_LAMB = max(5.0, 1500.0 / 1.1)
_INV = 1.0 / (1.0 + _LAMB)
_LANES = 16
_WIN = 128


def _sc_gather(cos_theta, phi_theta, t32, n_rows, n_cols):
    info = plsc.get_sparse_core_info()
    nw = info.num_cores * info.num_subcores
    bpw = n_rows // nw
    mesh = plsc.VectorSubcoreMesh(core_axis_name="c", subcore_axis_name="s")

    @functools.partial(
        pl.kernel,
        mesh=mesh,
        out_type=(
            jax.ShapeDtypeStruct((n_rows, 8, _WIN), jnp.float32),
            jax.ShapeDtypeStruct((n_rows, 8, _WIN), jnp.float32),
        ),
        scratch_types=[
            pltpu.VMEM((bpw,), jnp.int32),
            pltpu.SemaphoreType.DMA,
        ],
    )
    def gather_kernel(cos_hbm, phi_hbm, t_hbm, cwin_out, pwin_out, t_v, sem):
        wid = lax.axis_index("s") * info.num_cores + lax.axis_index("c")
        base = wid * bpw
        pltpu.sync_copy(t_hbm.at[pl.ds(base, bpw)], t_v)
        for src, dst in ((cos_hbm, cwin_out), (phi_hbm, pwin_out)):
            for g in range(0, bpw, _LANES):
                c0v = t_v[pl.ds(g, _LANES)] & ~(_WIN - 1)
                copies = []
                for j in range(_LANES):
                    r = base + g + j
                    c0 = pl.multiple_of(c0v[j], _WIN)
                    copies.append(pltpu.async_copy(
                        src.at[pl.ds((r // 8) * 8, 8), pl.ds(c0, _WIN)],
                        dst.at[r], sem))
                for cp in copies:
                    cp.wait()

    return gather_kernel(cos_theta, phi_theta, t32)


def _tc_body(cos_ref, cwin_ref, pwin_ref, tgt_ref, loss_ref, acc_ref, *, n_rows):
    i = pl.program_id(0)
    n = pl.num_programs(0)
    x = cos_ref[...]
    bm = jnp.max(x, axis=1, keepdims=True)
    s = jnp.sum(jnp.exp(x - bm), axis=1, keepdims=True)
    col = lax.broadcasted_iota(jnp.int32, x.shape, 1)
    cand = jnp.where(x == bm, col, jnp.int32(2**30))
    amax = jnp.min(cand, axis=1, keepdims=True)

    t = tgt_ref[...]
    rb = t.shape[0]
    row3 = lax.broadcasted_iota(jnp.int32, (rb, 8, _WIN), 0)
    sub3 = lax.broadcasted_iota(jnp.int32, (rb, 8, _WIN), 1)
    lane3 = lax.broadcasted_iota(jnp.int32, (rb, 8, _WIN), 2)
    fine = (sub3 == (row3 & 7)) & (lane3 == (t & (_WIN - 1))[:, :, None])
    ct = jnp.sum(jnp.where(fine, cwin_ref[...], 0.0), axis=(1, 2), keepdims=False)[:, None]
    pt = jnp.sum(jnp.where(fine, pwin_ref[...], 0.0), axis=(1, 2), keepdims=False)[:, None]
    v = ct + (pt - ct) * _INV
    s_adj = s - jnp.exp(ct - bm) + jnp.exp(v - bm)
    logpt = v - bm - jnp.log(s_adj)
    part_loss = -jnp.sum(logpt, axis=(0, 1), keepdims=True)
    part_acc = jnp.sum((amax == t).astype(jnp.float32), axis=(0, 1), keepdims=True)

    @pl.when(i == 0)
    def _init():
        loss_ref[...] = jnp.zeros((1, 1), jnp.float32)
        acc_ref[...] = jnp.zeros((1, 1), jnp.float32)

    loss_ref[...] += part_loss
    acc_ref[...] += part_acc

    @pl.when(i == n - 1)
    def _finish():
        scale = jnp.float32(1.0 / n_rows)
        loss_ref[...] *= scale
        acc_ref[...] *= scale


def _tc_pass(cos_theta, cwin, pwin, t32, row_block):
    n_rows, n_cols = cos_theta.shape
    grid = (n_rows // row_block,)
    body = functools.partial(_tc_body, n_rows=n_rows)
    return pl.pallas_call(
        body,
        grid=grid,
        in_specs=[
            pl.BlockSpec((row_block, n_cols), lambda i: (i, 0)),
            pl.BlockSpec((row_block, 8, _WIN), lambda i: (i, 0, 0)),
            pl.BlockSpec((row_block, 8, _WIN), lambda i: (i, 0, 0)),
            pl.BlockSpec((row_block, 1), lambda i: (i, 0)),
        ],
        out_specs=(
            pl.BlockSpec((1, 1), lambda i: (0, 0)),
            pl.BlockSpec((1, 1), lambda i: (0, 0)),
        ),
        out_shape=(
            jax.ShapeDtypeStruct((1, 1), jnp.float32),
            jax.ShapeDtypeStruct((1, 1), jnp.float32),
        ),
        compiler_params=pltpu.CompilerParams(
            dimension_semantics=("arbitrary",),
        ),
    )(cos_theta, cwin, pwin, t32)


def kernel(cos_theta, phi_theta, target):
    n_rows, n_cols = cos_theta.shape
    t32 = target.reshape(-1).astype(jnp.int32)
    cwin, pwin = _sc_gather(cos_theta, phi_theta, t32, n_rows, n_cols)
    loss, acc = _tc_pass(
        cos_theta,
        cwin,
        pwin,
        t32.reshape(n_rows, 1),
        row_block=8,
    )
    return loss[0, 0], acc[0, 0]


# speedup vs baseline: 1.9467x; 1.2405x over previous
import functools

import jax
import jax.numpy as jnp
from jax import lax
from jax.experimental import pallas as pl
from jax.experimental.pallas import tpu as pltpu
from jax.experimental.pallas import tpu_sc as plsc

_LAMB = max(5.0, 1500.0 / 1.1)
_INV = 1.0 / (1.0 + _LAMB)
_LANES = 16
_WIN = 128


def _sc_gather(cos_theta, phi_theta, t32, n_rows, n_cols):
    info = plsc.get_sparse_core_info()
    nw = info.num_cores * info.num_subcores
    bpw = n_rows // nw
    mesh = plsc.VectorSubcoreMesh(core_axis_name="c", subcore_axis_name="s")

    @functools.partial(
        pl.kernel,
        mesh=mesh,
        out_type=(
            jax.ShapeDtypeStruct((n_rows, 8, _WIN), jnp.float32),
            jax.ShapeDtypeStruct((n_rows, 8, _WIN), jnp.float32),
        ),
        scratch_types=[
            pltpu.VMEM((bpw,), jnp.int32),
            pltpu.SemaphoreType.DMA,
        ],
    )
    def gather_kernel(cos_hbm, phi_hbm, t_hbm, cwin_out, pwin_out, t_v, sem):
        wid = lax.axis_index("s") * info.num_cores + lax.axis_index("c")
        base = wid * bpw
        pltpu.sync_copy(t_hbm.at[pl.ds(base, bpw)], t_v)
        for src, dst in ((cos_hbm, cwin_out), (phi_hbm, pwin_out)):
            for g in range(0, bpw, _LANES):
                c0v = t_v[pl.ds(g, _LANES)] & ~(_WIN - 1)
                copies = []
                for j in range(_LANES):
                    r = base + g + j
                    c0 = pl.multiple_of(c0v[j], _WIN)
                    copies.append(pltpu.async_copy(
                        src.at[pl.ds((r // 8) * 8, 8), pl.ds(c0, _WIN)],
                        dst.at[r], sem))
                for cp in copies:
                    cp.wait()

    return gather_kernel(cos_theta, phi_theta, t32)


def _stream_body(cos_ref, m_ref, s_ref, amax_ref):
    x = cos_ref[...]
    bm = jnp.max(x, axis=1, keepdims=True)
    s = jnp.sum(jnp.exp(x - bm), axis=1, keepdims=True)
    col = lax.broadcasted_iota(jnp.int32, x.shape, 1)
    cand = jnp.where(x == bm, col, jnp.int32(2**30))
    amax = jnp.min(cand, axis=1, keepdims=True)
    m_ref[...] = bm
    s_ref[...] = s
    amax_ref[...] = amax


def _tc_stream(cos_theta, row_block):
    n_rows, n_cols = cos_theta.shape
    grid = (n_rows // row_block,)
    return pl.pallas_call(
        _stream_body,
        grid=grid,
        in_specs=[pl.BlockSpec((row_block, n_cols), lambda i: (i, 0))],
        out_specs=(
            pl.BlockSpec((row_block, 1), lambda i: (i, 0)),
            pl.BlockSpec((row_block, 1), lambda i: (i, 0)),
            pl.BlockSpec((row_block, 1), lambda i: (i, 0)),
        ),
        out_shape=(
            jax.ShapeDtypeStruct((n_rows, 1), jnp.float32),
            jax.ShapeDtypeStruct((n_rows, 1), jnp.float32),
            jax.ShapeDtypeStruct((n_rows, 1), jnp.int32),
        ),
        compiler_params=pltpu.CompilerParams(
            dimension_semantics=("arbitrary",),
        ),
    )(cos_theta)


def _epilogue_body(m_ref, s_ref, amax_ref, tgt_ref, cwin_ref, pwin_ref,
                   loss_ref, acc_ref):
    bm = m_ref[...]
    s = s_ref[...]
    amax = amax_ref[...]
    t = tgt_ref[...]
    n = t.shape[0]
    row3 = lax.broadcasted_iota(jnp.int32, (n, 8, _WIN), 0)
    sub3 = lax.broadcasted_iota(jnp.int32, (n, 8, _WIN), 1)
    lane3 = lax.broadcasted_iota(jnp.int32, (n, 8, _WIN), 2)
    fine = (sub3 == (row3 & 7)) & (lane3 == (t & (_WIN - 1))[:, :, None])
    ct = jnp.sum(jnp.where(fine, cwin_ref[...], 0.0), axis=(1, 2))[:, None]
    pt = jnp.sum(jnp.where(fine, pwin_ref[...], 0.0), axis=(1, 2))[:, None]
    v = ct + (pt - ct) * _INV
    s_adj = s - jnp.exp(ct - bm) + jnp.exp(v - bm)
    logpt = v - bm - jnp.log(s_adj)
    scale = jnp.float32(1.0 / n)
    loss_ref[...] = -jnp.sum(logpt, axis=(0, 1), keepdims=True) * scale
    acc_ref[...] = jnp.sum((amax == t).astype(jnp.float32),
                           axis=(0, 1), keepdims=True) * scale


def _tc_epilogue(m, s, amax, t32, cwin, pwin):
    n_rows = m.shape[0]
    return pl.pallas_call(
        _epilogue_body,
        in_specs=[
            pl.BlockSpec((n_rows, 1), lambda: (0, 0)),
            pl.BlockSpec((n_rows, 1), lambda: (0, 0)),
            pl.BlockSpec((n_rows, 1), lambda: (0, 0)),
            pl.BlockSpec((n_rows, 1), lambda: (0, 0)),
            pl.BlockSpec((n_rows, 8, _WIN), lambda: (0, 0, 0)),
            pl.BlockSpec((n_rows, 8, _WIN), lambda: (0, 0, 0)),
        ],
        out_specs=(
            pl.BlockSpec((1, 1), lambda: (0, 0)),
            pl.BlockSpec((1, 1), lambda: (0, 0)),
        ),
        out_shape=(
            jax.ShapeDtypeStruct((1, 1), jnp.float32),
            jax.ShapeDtypeStruct((1, 1), jnp.float32),
        ),
    )(m, s, amax, t32, cwin, pwin)


def kernel(cos_theta, phi_theta, target):
    n_rows, n_cols = cos_theta.shape
    t32 = target.reshape(-1).astype(jnp.int32)
    cwin, pwin = _sc_gather(cos_theta, phi_theta, t32, n_rows, n_cols)
    m, s, amax = _tc_stream(cos_theta, row_block=8)
    loss, acc = _tc_epilogue(m, s, amax, t32.reshape(n_rows, 1), cwin, pwin)
    return loss[0, 0], acc[0, 0]


# speedup vs baseline: 1.9496x; 1.0015x over previous
import functools

import jax
import jax.numpy as jnp
from jax import lax
from jax.experimental import pallas as pl
from jax.experimental.pallas import tpu as pltpu
from jax.experimental.pallas import tpu_sc as plsc

_LAMB = max(5.0, 1500.0 / 1.1)
_INV = 1.0 / (1.0 + _LAMB)
_LANES = 16
_WIN = 128


def _sc_gather(cos_theta, phi_theta, t32, n_rows, n_cols):
    info = plsc.get_sparse_core_info()
    nw = info.num_cores * info.num_subcores
    bpw = n_rows // nw
    mesh = plsc.VectorSubcoreMesh(core_axis_name="c", subcore_axis_name="s")

    @functools.partial(
        pl.kernel,
        mesh=mesh,
        out_type=(
            jax.ShapeDtypeStruct((n_rows, 8, _WIN), jnp.float32),
            jax.ShapeDtypeStruct((n_rows, 8, _WIN), jnp.float32),
        ),
        scratch_types=[
            pltpu.VMEM((bpw,), jnp.int32),
            pltpu.SemaphoreType.DMA,
        ],
        compiler_params=pltpu.CompilerParams(use_tc_tiling_on_sc=True),
    )
    def gather_kernel(cos_hbm, phi_hbm, t_hbm, cwin_out, pwin_out, t_v, sem):
        wid = lax.axis_index("s") * info.num_cores + lax.axis_index("c")
        base = wid * bpw
        pltpu.sync_copy(t_hbm.at[pl.ds(base, bpw)], t_v)
        for src, dst in ((cos_hbm, cwin_out), (phi_hbm, pwin_out)):
            for g in range(0, bpw, _LANES):
                c0v = t_v[pl.ds(g, _LANES)] & ~(_WIN - 1)
                copies = []
                for j in range(_LANES):
                    r = base + g + j
                    c0 = pl.multiple_of(c0v[j], _WIN)
                    copies.append(pltpu.async_copy(
                        src.at[pl.ds((r // 8) * 8, 8), pl.ds(c0, _WIN)],
                        dst.at[r], sem))
                for cp in copies:
                    cp.wait()

    return gather_kernel(cos_theta, phi_theta, t32)


def _stream_body(cos_ref, m_ref, s_ref, amax_ref):
    x = cos_ref[...]
    bm = jnp.max(x, axis=1, keepdims=True)
    s = jnp.sum(jnp.exp(x - bm), axis=1, keepdims=True)
    col = lax.broadcasted_iota(jnp.int32, x.shape, 1)
    cand = jnp.where(x == bm, col, jnp.int32(2**30))
    amax = jnp.min(cand, axis=1, keepdims=True)
    m_ref[...] = bm
    s_ref[...] = s
    amax_ref[...] = amax


def _tc_stream(cos_theta, row_block):
    n_rows, n_cols = cos_theta.shape
    grid = (n_rows // row_block,)
    return pl.pallas_call(
        _stream_body,
        grid=grid,
        in_specs=[pl.BlockSpec((row_block, n_cols), lambda i: (i, 0))],
        out_specs=(
            pl.BlockSpec((row_block, 1), lambda i: (i, 0)),
            pl.BlockSpec((row_block, 1), lambda i: (i, 0)),
            pl.BlockSpec((row_block, 1), lambda i: (i, 0)),
        ),
        out_shape=(
            jax.ShapeDtypeStruct((n_rows, 1), jnp.float32),
            jax.ShapeDtypeStruct((n_rows, 1), jnp.float32),
            jax.ShapeDtypeStruct((n_rows, 1), jnp.int32),
        ),
        compiler_params=pltpu.CompilerParams(
            dimension_semantics=("arbitrary",),
        ),
    )(cos_theta)


def _epilogue_body(m_ref, s_ref, amax_ref, tgt_ref, cwin_ref, pwin_ref,
                   loss_ref, acc_ref):
    bm = m_ref[...]
    s = s_ref[...]
    amax = amax_ref[...]
    t = tgt_ref[...]
    n = t.shape[0]
    row3 = lax.broadcasted_iota(jnp.int32, (n, 8, _WIN), 0)
    sub3 = lax.broadcasted_iota(jnp.int32, (n, 8, _WIN), 1)
    lane3 = lax.broadcasted_iota(jnp.int32, (n, 8, _WIN), 2)
    fine = (sub3 == (row3 & 7)) & (lane3 == (t & (_WIN - 1))[:, :, None])
    ct = jnp.sum(jnp.where(fine, cwin_ref[...], 0.0), axis=(1, 2))[:, None]
    pt = jnp.sum(jnp.where(fine, pwin_ref[...], 0.0), axis=(1, 2))[:, None]
    v = ct + (pt - ct) * _INV
    s_adj = s - jnp.exp(ct - bm) + jnp.exp(v - bm)
    logpt = v - bm - jnp.log(s_adj)
    scale = jnp.float32(1.0 / n)
    loss_ref[...] = -jnp.sum(logpt, axis=(0, 1), keepdims=True) * scale
    acc_ref[...] = jnp.sum((amax == t).astype(jnp.float32),
                           axis=(0, 1), keepdims=True) * scale


def _tc_epilogue(m, s, amax, t32, cwin, pwin):
    n_rows = m.shape[0]
    return pl.pallas_call(
        _epilogue_body,
        in_specs=[
            pl.BlockSpec((n_rows, 1), lambda: (0, 0)),
            pl.BlockSpec((n_rows, 1), lambda: (0, 0)),
            pl.BlockSpec((n_rows, 1), lambda: (0, 0)),
            pl.BlockSpec((n_rows, 1), lambda: (0, 0)),
            pl.BlockSpec((n_rows, 8, _WIN), lambda: (0, 0, 0)),
            pl.BlockSpec((n_rows, 8, _WIN), lambda: (0, 0, 0)),
        ],
        out_specs=(
            pl.BlockSpec((1, 1), lambda: (0, 0)),
            pl.BlockSpec((1, 1), lambda: (0, 0)),
        ),
        out_shape=(
            jax.ShapeDtypeStruct((1, 1), jnp.float32),
            jax.ShapeDtypeStruct((1, 1), jnp.float32),
        ),
    )(m, s, amax, t32, cwin, pwin)


def kernel(cos_theta, phi_theta, target):
    n_rows, n_cols = cos_theta.shape
    t32 = target.reshape(-1).astype(jnp.int32)
    cwin, pwin = _sc_gather(cos_theta, phi_theta, t32, n_rows, n_cols)
    m, s, amax = _tc_stream(cos_theta, row_block=8)
    loss, acc = _tc_epilogue(m, s, amax, t32.reshape(n_rows, 1), cwin, pwin)
    return loss[0, 0], acc[0, 0]
